# baseline (speedup 1.0000x reference)
import jax
import jax.numpy as jnp
from jax import lax
from jax.experimental import pallas as pl
from jax.experimental.pallas import tpu as pltpu
from jax.experimental.pallas import tpu_sc as plsc

_K = 32
_H = 8192
_D = 768
_RT_ENC = 256
_HT = 1024
_RT_THR = 128
_RT_DEC = 256


_NV = _H // 16
_CAP = 2048
_CAPP = 2053
_NGRP = 1


def _key16(x):
    bits = lax.bitcast_convert_type(x, jnp.int32)
    return jnp.where(x >= 0.0, bits, ~(bits) ^ jnp.int32(-2147483648))


def _sc_threshold_body(z_hbm, t_hbm, rowbuf, hist, sacc, cand, tbuf, sem):
    nc = 2
    wid = lax.axis_index("s") * nc + lax.axis_index("c")
    base = wid * (_NGRP * 16)
    lane = lax.iota(jnp.int32, 16)
    zero16 = jnp.zeros((16,), jnp.int32)
    one16 = jnp.ones((16,), jnp.int32)
    kv = jnp.full((16,), _K, jnp.int32)

    def zh(b, _):
        hist[pl.ds(b * 16, 16)] = zero16
        return 0
    lax.fori_loop(0, 256, zh, 0, unroll=8)

    def walk_and_search(k_remv):
        def walk(i, acc):
            bb = 255 - i
            acc = acc + hist[pl.ds(bb * 16, 16)]
            sacc[pl.ds(bb * 16, 16)] = acc
            hist[pl.ds(bb * 16, 16)] = zero16
            return acc
        plsc.parallel_loop(0, 256, unroll=8, carry=zero16)(walk)
        bv = zero16
        for sbit in (128, 64, 32, 16, 8, 4, 2, 1):
            cb = bv + sbit
            tot = plsc.load_gather(sacc, [cb * 16 + lane])
            bv = jnp.where(tot >= k_remv, cb, bv)
        abv = jnp.minimum(bv + 1, 255)
        na = plsc.load_gather(sacc, [abv * 16 + lane])
        n_above = jnp.where(bv >= 255, zero16, na)
        return bv, k_remv - n_above

    def group_body(g):
        r0 = base + g * 16

        for ch in range(4):
            pltpu.async_copy(
                z_hbm.at[pl.ds(r0, 16), pl.ds(ch * 2048, 2048)], rowbuf, sem
            ).wait()

            @plsc.parallel_loop(0, 2048, unroll=8)
            def _pa(c):
                cc = (jnp.zeros((16,), jnp.int32) + c + lane) & 2047
                x = plsc.load_gather(rowbuf, [lane, cc])
                s = _key16(x)
                b = lax.shift_right_arithmetic(s, 24) + 128
                plsc.addupdate_scatter(hist, [b * 16 + lane], one16)

        b1v, k2v = walk_and_search(kv)

        offv = zero16

        for ch in range(4):
            pltpu.async_copy(
                z_hbm.at[pl.ds(r0, 16), pl.ds(ch * 2048, 2048)], rowbuf, sem
            ).wait()

            def pc(c, offv):
                cc = (jnp.zeros((16,), jnp.int32) + c + lane) & 2047
                x = plsc.load_gather(rowbuf, [lane, cc])
                s = _key16(x)
                b = lax.shift_right_arithmetic(s, 24) + 128
                m = (b == b1v) & (offv < _CAP)
                plsc.store_scatter(cand, [lane * _CAPP + offv], s, mask=m)
                return offv + m.astype(jnp.int32)
            offv = plsc.parallel_loop(0, 2048, unroll=8, carry=offv)(pc)

        mo = zero16
        for sbit in (1024, 512, 256, 128, 64, 32, 16, 8, 4, 2, 1):
            test = mo + sbit
            anyv = plsc.all_reduce_population_count(offv >= test)
            mo = jnp.where(anyv > 0, test, mo)
        mos = jnp.sum(mo) // 16

        def lvl_pass(shift, match2, b2v, match3, b3v):
            def pl_body(i):
                iv = jnp.zeros((16,), jnp.int32) + i
                sv = plsc.load_gather(cand, [lane * _CAPP + iv])
                m = iv < offv
                if match2:
                    m = m & ((lax.shift_right_logical(sv, 16) & 0xFF) == b2v)
                if match3:
                    m = m & ((lax.shift_right_logical(sv, 8) & 0xFF) == b3v)
                b = lax.shift_right_logical(sv, shift) & 0xFF
                plsc.addupdate_scatter(hist, [b * 16 + lane], one16, mask=m)
            plsc.parallel_loop(0, mos, unroll=2)(pl_body)

        lvl_pass(16, False, zero16, False, zero16)
        b2v, k3v = walk_and_search(k2v)
        lvl_pass(8, True, b2v, False, zero16)
        b3v, k4v = walk_and_search(k3v)
        lvl_pass(0, True, b2v, True, b3v)
        b4v, _ = walk_and_search(k4v)

        keyv = (lax.shift_left(b1v - 128, 24) | lax.shift_left(b2v, 16)
                | lax.shift_left(b3v, 8) | b4v)
        tbv = jnp.where(keyv >= 0, keyv, ~(keyv ^ jnp.int32(-2147483648)))
        tfv = lax.bitcast_convert_type(tbv, jnp.float32)
        plsc.store_scatter(tbuf, [g * 16 + lane], tfv)

    for g in range(_NGRP):
        group_body(g)
    pltpu.sync_copy(tbuf, t_hbm.at[pl.ds(base, _NGRP * 16)])


def make_sc_threshold(n):
    assert n == _NGRP * 512
    mesh = plsc.VectorSubcoreMesh(core_axis_name="c", subcore_axis_name="s")
    return pl.kernel(
        _sc_threshold_body,
        out_type=jax.ShapeDtypeStruct((n,), jnp.float32),
        mesh=mesh,
        compiler_params=pltpu.CompilerParams(needs_layout_passes=False),
        scratch_types=[
            pltpu.VMEM((16, 2048), jnp.float32),
            pltpu.VMEM((4096,), jnp.int32),
            pltpu.VMEM((4096,), jnp.int32),
            pltpu.VMEM((16 * _CAPP,), jnp.int32),
            pltpu.VMEM((_NGRP * 16,), jnp.float32),
            pltpu.SemaphoreType.DMA,
        ],
    )


def _encode_body(x_ref, we_ref, be_ref, z_ref):
    z_ref[...] = jax.lax.dot_general(
        x_ref[...], we_ref[...],
        dimension_numbers=(((1,), (1,)), ((), ())),
        preferred_element_type=jnp.float32,
    ) + be_ref[...]


def _decode_body(z_ref, t_ref, wd_ref, bd_ref, zs_ref, xh_ref):
    j = pl.program_id(1)
    z = z_ref[...]
    zs = jnp.where(z >= t_ref[...], z, 0.0)
    zs_ref[...] = zs
    part = jax.lax.dot_general(
        zs, wd_ref[...],
        dimension_numbers=(((1,), (1,)), ((), ())),
        preferred_element_type=jnp.float32,
    )

    @pl.when(j == 0)
    def _():
        xh_ref[...] = part + bd_ref[...]

    @pl.when(j != 0)
    def _():
        xh_ref[...] += part


def kernel(x, W_enc, b_enc, W_dec, b_dec):
    be = b_enc.reshape(1, _H)
    bd = b_dec.reshape(1, _D)
    chunks = [
        _chunk(x[c * 512:(c + 1) * 512], W_enc, be, W_dec, bd)
        for c in range(x.shape[0] // 512)
    ]
    zs = jnp.concatenate([c[0] for c in chunks], axis=0)
    xh = jnp.concatenate([c[1] for c in chunks], axis=0)
    return (zs, xh)


def _chunk(x, W_enc, be, W_dec, bd):
    n = x.shape[0]

    z = pl.pallas_call(
        _encode_body,
        grid=(n // _RT_ENC, _H // _HT),
        in_specs=[
            pl.BlockSpec((_RT_ENC, _D), lambda i, j: (i, 0)),
            pl.BlockSpec((_HT, _D), lambda i, j: (j, 0)),
            pl.BlockSpec((1, _HT), lambda i, j: (0, j)),
        ],
        out_specs=pl.BlockSpec((_RT_ENC, _HT), lambda i, j: (i, j)),
        out_shape=jax.ShapeDtypeStruct((n, _H), jnp.float32),
        compiler_params=pltpu.CompilerParams(
            dimension_semantics=("parallel", "arbitrary"),
        ),
    )(x, W_enc, be)

    t = make_sc_threshold(n)(z).reshape(n, 1)

    zs, xh = pl.pallas_call(
        _decode_body,
        grid=(n // _RT_DEC, _H // _HT),
        in_specs=[
            pl.BlockSpec((_RT_DEC, _HT), lambda i, j: (i, j)),
            pl.BlockSpec((_RT_DEC, 1), lambda i, j: (i, 0)),
            pl.BlockSpec((_D, _HT), lambda i, j: (0, j)),
            pl.BlockSpec((1, _D), lambda i, j: (0, 0)),
        ],
        out_specs=[
            pl.BlockSpec((_RT_DEC, _HT), lambda i, j: (i, j)),
            pl.BlockSpec((_RT_DEC, _D), lambda i, j: (i, 0)),
        ],
        out_shape=[
            jax.ShapeDtypeStruct((n, _H), jnp.float32),
            jax.ShapeDtypeStruct((n, _D), jnp.float32),
        ],
        compiler_params=pltpu.CompilerParams(
            dimension_semantics=("parallel", "arbitrary"),
        ),
    )(z, t, W_dec, bd)
    return (zs, xh)


# speedup vs baseline: 1.0513x; 1.0513x over previous
import jax
import jax.numpy as jnp
from jax import lax
from jax.experimental import pallas as pl
from jax.experimental.pallas import tpu as pltpu
from jax.experimental.pallas import tpu_sc as plsc

_K = 32
_H = 8192
_D = 768
_RT_ENC = 256
_HT = 1024
_RT_THR = 128
_RT_DEC = 256


_NV = _H // 16
_CAP = 2048
_CAPP = 2053
_NGRP = 2


def _key16(x):
    bits = lax.bitcast_convert_type(x, jnp.int32)
    return jnp.where(x >= 0.0, bits, ~(bits) ^ jnp.int32(-2147483648))


def _sc_threshold_body(z_hbm, t_hbm, rowbuf, hist, sacc, cand, tbuf, sem):
    nc = 2
    wid = lax.axis_index("s") * nc + lax.axis_index("c")
    base = wid * (_NGRP * 16)
    lane = lax.iota(jnp.int32, 16)
    zero16 = jnp.zeros((16,), jnp.int32)
    one16 = jnp.ones((16,), jnp.int32)
    kv = jnp.full((16,), _K, jnp.int32)

    def zh(b, _):
        hist[pl.ds(b * 16, 16)] = zero16
        return 0
    lax.fori_loop(0, 256, zh, 0, unroll=8)

    def walk_and_search(k_remv):
        def walk(i, acc):
            bb = 255 - i
            acc = acc + hist[pl.ds(bb * 16, 16)]
            sacc[pl.ds(bb * 16, 16)] = acc
            hist[pl.ds(bb * 16, 16)] = zero16
            return acc
        plsc.parallel_loop(0, 256, unroll=8, carry=zero16)(walk)
        bv = zero16
        for sbit in (128, 64, 32, 16, 8, 4, 2, 1):
            cb = bv + sbit
            tot = plsc.load_gather(sacc, [cb * 16 + lane])
            bv = jnp.where(tot >= k_remv, cb, bv)
        abv = jnp.minimum(bv + 1, 255)
        na = plsc.load_gather(sacc, [abv * 16 + lane])
        n_above = jnp.where(bv >= 255, zero16, na)
        return bv, k_remv - n_above

    def group_body(g):
        r0 = base + g * 16

        for ch in range(4):
            pltpu.async_copy(
                z_hbm.at[pl.ds(r0, 16), pl.ds(ch * 2048, 2048)], rowbuf, sem
            ).wait()

            @plsc.parallel_loop(0, 2048, unroll=16)
            def _pa(c):
                cc = (jnp.zeros((16,), jnp.int32) + c + lane) & 2047
                x = plsc.load_gather(rowbuf, [lane, cc])
                s = _key16(x)
                b = lax.shift_right_arithmetic(s, 24) + 128
                plsc.addupdate_scatter(hist, [b * 16 + lane], one16)

        b1v, k2v = walk_and_search(kv)

        offv = zero16

        for ch in range(4):
            pltpu.async_copy(
                z_hbm.at[pl.ds(r0, 16), pl.ds(ch * 2048, 2048)], rowbuf, sem
            ).wait()

            def pc(c, offv):
                cc = (jnp.zeros((16,), jnp.int32) + c + lane) & 2047
                x = plsc.load_gather(rowbuf, [lane, cc])
                s = _key16(x)
                b = lax.shift_right_arithmetic(s, 24) + 128
                m = (b == b1v) & (offv < _CAP)
                plsc.store_scatter(cand, [lane * _CAPP + offv], s, mask=m)
                return offv + m.astype(jnp.int32)
            offv = plsc.parallel_loop(0, 2048, unroll=16, carry=offv)(pc)

        mo = zero16
        for sbit in (1024, 512, 256, 128, 64, 32, 16, 8, 4, 2, 1):
            test = mo + sbit
            anyv = plsc.all_reduce_population_count(offv >= test)
            mo = jnp.where(anyv > 0, test, mo)
        mos = jnp.sum(mo) // 16

        def lvl_pass(shift, match2, b2v, match3, b3v):
            def pl_body(i):
                iv = jnp.zeros((16,), jnp.int32) + i
                sv = plsc.load_gather(cand, [lane * _CAPP + iv])
                m = iv < offv
                if match2:
                    m = m & ((lax.shift_right_logical(sv, 16) & 0xFF) == b2v)
                if match3:
                    m = m & ((lax.shift_right_logical(sv, 8) & 0xFF) == b3v)
                b = lax.shift_right_logical(sv, shift) & 0xFF
                plsc.addupdate_scatter(hist, [b * 16 + lane], one16, mask=m)
            plsc.parallel_loop(0, mos, unroll=2)(pl_body)

        lvl_pass(16, False, zero16, False, zero16)
        b2v, k3v = walk_and_search(k2v)
        lvl_pass(8, True, b2v, False, zero16)
        b3v, k4v = walk_and_search(k3v)
        lvl_pass(0, True, b2v, True, b3v)
        b4v, _ = walk_and_search(k4v)

        keyv = (lax.shift_left(b1v - 128, 24) | lax.shift_left(b2v, 16)
                | lax.shift_left(b3v, 8) | b4v)
        tbv = jnp.where(keyv >= 0, keyv, ~(keyv ^ jnp.int32(-2147483648)))
        tfv = lax.bitcast_convert_type(tbv, jnp.float32)
        plsc.store_scatter(tbuf, [g * 16 + lane], tfv)

    for g in range(_NGRP):
        group_body(g)
    pltpu.sync_copy(tbuf, t_hbm.at[pl.ds(base, _NGRP * 16)])


def make_sc_threshold(n):
    assert n == _NGRP * 512
    mesh = plsc.VectorSubcoreMesh(core_axis_name="c", subcore_axis_name="s")
    return pl.kernel(
        _sc_threshold_body,
        out_type=jax.ShapeDtypeStruct((n,), jnp.float32),
        mesh=mesh,
        compiler_params=pltpu.CompilerParams(needs_layout_passes=False),
        scratch_types=[
            pltpu.VMEM((16, 2048), jnp.float32),
            pltpu.VMEM((4096,), jnp.int32),
            pltpu.VMEM((4096,), jnp.int32),
            pltpu.VMEM((16 * _CAPP,), jnp.int32),
            pltpu.VMEM((_NGRP * 16,), jnp.float32),
            pltpu.SemaphoreType.DMA,
        ],
    )


def _encode_body(x_ref, we_ref, be_ref, z_ref):
    z_ref[...] = jax.lax.dot_general(
        x_ref[...], we_ref[...],
        dimension_numbers=(((1,), (1,)), ((), ())),
        preferred_element_type=jnp.float32,
    ) + be_ref[...]


def _decode_body(z_ref, t_ref, wd_ref, bd_ref, zs_ref, xh_ref):
    j = pl.program_id(1)
    z = z_ref[...]
    zs = jnp.where(z >= t_ref[...], z, 0.0)
    zs_ref[...] = zs
    part = jax.lax.dot_general(
        zs.astype(jnp.bfloat16), wd_ref[...].astype(jnp.bfloat16),
        dimension_numbers=(((1,), (1,)), ((), ())),
        preferred_element_type=jnp.float32,
    )

    @pl.when(j == 0)
    def _():
        xh_ref[...] = part + bd_ref[...]

    @pl.when(j != 0)
    def _():
        xh_ref[...] += part


def kernel(x, W_enc, b_enc, W_dec, b_dec):
    be = b_enc.reshape(1, _H)
    bd = b_dec.reshape(1, _D)
    return _chunk(x, W_enc, be, W_dec, bd)


def _chunk(x, W_enc, be, W_dec, bd):
    n = x.shape[0]

    z = pl.pallas_call(
        _encode_body,
        grid=(n // _RT_ENC, _H // _HT),
        in_specs=[
            pl.BlockSpec((_RT_ENC, _D), lambda i, j: (i, 0)),
            pl.BlockSpec((_HT, _D), lambda i, j: (j, 0)),
            pl.BlockSpec((1, _HT), lambda i, j: (0, j)),
        ],
        out_specs=pl.BlockSpec((_RT_ENC, _HT), lambda i, j: (i, j)),
        out_shape=jax.ShapeDtypeStruct((n, _H), jnp.float32),
        compiler_params=pltpu.CompilerParams(
            dimension_semantics=("parallel", "arbitrary"),
        ),
    )(x, W_enc, be)

    t = make_sc_threshold(n)(z).reshape(n, 1)

    zs, xh = pl.pallas_call(
        _decode_body,
        grid=(n // _RT_DEC, _H // _HT),
        in_specs=[
            pl.BlockSpec((_RT_DEC, _HT), lambda i, j: (i, j)),
            pl.BlockSpec((_RT_DEC, 1), lambda i, j: (i, 0)),
            pl.BlockSpec((_D, _HT), lambda i, j: (0, j)),
            pl.BlockSpec((1, _D), lambda i, j: (0, 0)),
        ],
        out_specs=[
            pl.BlockSpec((_RT_DEC, _HT), lambda i, j: (i, j)),
            pl.BlockSpec((_RT_DEC, _D), lambda i, j: (i, 0)),
        ],
        out_shape=[
            jax.ShapeDtypeStruct((n, _H), jnp.float32),
            jax.ShapeDtypeStruct((n, _D), jnp.float32),
        ],
        compiler_params=pltpu.CompilerParams(
            dimension_semantics=("parallel", "arbitrary"),
        ),
    )(z, t, W_dec, bd)
    return (zs, xh)


# speedup vs baseline: 1.1317x; 1.0765x over previous
import jax
import jax.numpy as jnp
from jax import lax
from jax.experimental import pallas as pl
from jax.experimental.pallas import tpu as pltpu
from jax.experimental.pallas import tpu_sc as plsc

_K = 32
_H = 8192
_D = 768
_RT_ENC = 256
_HT = 1024
_RT_THR = 128
_RT_DEC = 256


_NV = _H // 16
_CAP = 2048
_CAPP = 2053
_NGRP = 2


def _key16(x):
    bits = lax.bitcast_convert_type(x, jnp.int32)
    return jnp.where(x >= 0.0, bits, ~(bits) ^ jnp.int32(-2147483648))


def _sc_threshold_body(z_hbm, t_hbm, rowbuf, rowbuf2, hist, sacc, cand, tbuf, sem, sem2):
    nc = 2
    wid = lax.axis_index("s") * nc + lax.axis_index("c")
    base = wid * (_NGRP * 16)
    lane = lax.iota(jnp.int32, 16)
    zero16 = jnp.zeros((16,), jnp.int32)
    one16 = jnp.ones((16,), jnp.int32)
    kv = jnp.full((16,), _K, jnp.int32)

    def zh(b, _):
        hist[pl.ds(b * 16, 16)] = zero16
        return 0
    lax.fori_loop(0, 256, zh, 0, unroll=8)

    def walk_and_search(k_remv):
        def walk(i, acc):
            bb = 255 - i
            acc = acc + hist[pl.ds(bb * 16, 16)]
            sacc[pl.ds(bb * 16, 16)] = acc
            hist[pl.ds(bb * 16, 16)] = zero16
            return acc
        plsc.parallel_loop(0, 256, unroll=8, carry=zero16)(walk)
        bv = zero16
        for sbit in (128, 64, 32, 16, 8, 4, 2, 1):
            cb = bv + sbit
            tot = plsc.load_gather(sacc, [cb * 16 + lane])
            bv = jnp.where(tot >= k_remv, cb, bv)
        abv = jnp.minimum(bv + 1, 255)
        na = plsc.load_gather(sacc, [abv * 16 + lane])
        n_above = jnp.where(bv >= 255, zero16, na)
        return bv, k_remv - n_above

    def group_body(g):
        r0 = base + g * 16

        bufs = (rowbuf, rowbuf2)
        sems = (sem, sem2)

        def start(ch):
            return pltpu.async_copy(
                z_hbm.at[pl.ds(r0, 16), pl.ds(ch * 2048, 2048)],
                bufs[ch % 2], sems[ch % 2])

        cp = start(0)
        for ch in range(4):
            nxt = start(ch + 1) if ch < 3 else None
            cp.wait()
            buf = bufs[ch % 2]

            @plsc.parallel_loop(0, 2048, unroll=8)
            def _pa(c, buf=buf):
                cc = (jnp.zeros((16,), jnp.int32) + c + lane) & 2047
                x = plsc.load_gather(buf, [lane, cc])
                s = _key16(x)
                b = lax.shift_right_arithmetic(s, 24) + 128
                plsc.addupdate_scatter(hist, [b * 16 + lane], one16)
            cp = nxt

        b1v, k2v = walk_and_search(kv)

        offv = zero16

        cp = start(0)
        for ch in range(4):
            nxt = start(ch + 1) if ch < 3 else None
            cp.wait()
            buf = bufs[ch % 2]

            def pc(c, offv):
                cc = (jnp.zeros((16,), jnp.int32) + c + lane) & 2047
                x = plsc.load_gather(buf, [lane, cc])
                s = _key16(x)
                b = lax.shift_right_arithmetic(s, 24) + 128
                m = (b == b1v) & (offv < _CAP)
                plsc.store_scatter(cand, [lane * _CAPP + offv], s, mask=m)
                return offv + m.astype(jnp.int32)
            offv = plsc.parallel_loop(0, 2048, unroll=8, carry=offv)(pc)
            cp = nxt

        mo = zero16
        for sbit in (1024, 512, 256, 128, 64, 32, 16, 8, 4, 2, 1):
            test = mo + sbit
            anyv = plsc.all_reduce_population_count(offv >= test)
            mo = jnp.where(anyv > 0, test, mo)
        mos = jnp.sum(mo) // 16

        def lvl_pass(shift, match2, b2v, match3, b3v):
            def pl_body(i):
                iv = jnp.zeros((16,), jnp.int32) + i
                sv = plsc.load_gather(cand, [lane * _CAPP + iv])
                m = iv < offv
                if match2:
                    m = m & ((lax.shift_right_logical(sv, 16) & 0xFF) == b2v)
                if match3:
                    m = m & ((lax.shift_right_logical(sv, 8) & 0xFF) == b3v)
                b = lax.shift_right_logical(sv, shift) & 0xFF
                plsc.addupdate_scatter(hist, [b * 16 + lane], one16, mask=m)
            plsc.parallel_loop(0, mos, unroll=2)(pl_body)

        lvl_pass(16, False, zero16, False, zero16)
        b2v, k3v = walk_and_search(k2v)
        lvl_pass(8, True, b2v, False, zero16)
        b3v, k4v = walk_and_search(k3v)
        lvl_pass(0, True, b2v, True, b3v)
        b4v, _ = walk_and_search(k4v)

        keyv = (lax.shift_left(b1v - 128, 24) | lax.shift_left(b2v, 16)
                | lax.shift_left(b3v, 8) | b4v)
        tbv = jnp.where(keyv >= 0, keyv, ~(keyv ^ jnp.int32(-2147483648)))
        tfv = lax.bitcast_convert_type(tbv, jnp.float32)
        plsc.store_scatter(tbuf, [g * 16 + lane], tfv)

    for g in range(_NGRP):
        group_body(g)
    pltpu.sync_copy(tbuf, t_hbm.at[pl.ds(base, _NGRP * 16)])


def make_sc_threshold(n):
    assert n == _NGRP * 512
    mesh = plsc.VectorSubcoreMesh(core_axis_name="c", subcore_axis_name="s")
    return pl.kernel(
        _sc_threshold_body,
        out_type=jax.ShapeDtypeStruct((n,), jnp.float32),
        mesh=mesh,
        compiler_params=pltpu.CompilerParams(needs_layout_passes=False),
        scratch_types=[
            pltpu.VMEM((16, 2048), jnp.float32),
            pltpu.VMEM((16, 2048), jnp.float32),
            pltpu.VMEM((4096,), jnp.int32),
            pltpu.VMEM((4096,), jnp.int32),
            pltpu.VMEM((16 * _CAPP,), jnp.int32),
            pltpu.VMEM((_NGRP * 16,), jnp.float32),
            pltpu.SemaphoreType.DMA,
            pltpu.SemaphoreType.DMA,
        ],
    )


def _encode_body(x_ref, we_ref, be_ref, z_ref):
    z_ref[...] = jax.lax.dot_general(
        x_ref[...], we_ref[...],
        dimension_numbers=(((1,), (1,)), ((), ())),
        preferred_element_type=jnp.float32,
    ) + be_ref[...]


def _decode_body(z_ref, t_ref, wd_ref, bd_ref, zs_ref, xh_ref):
    j = pl.program_id(1)
    z = z_ref[...]
    zs = jnp.where(z >= t_ref[...], z, 0.0)
    zs_ref[...] = zs
    part = jax.lax.dot_general(
        zs, wd_ref[...],
        dimension_numbers=(((1,), (1,)), ((), ())),
        preferred_element_type=jnp.float32,
    )

    @pl.when(j == 0)
    def _():
        xh_ref[...] = part + bd_ref[...]

    @pl.when(j != 0)
    def _():
        xh_ref[...] += part


def kernel(x, W_enc, b_enc, W_dec, b_dec):
    be = b_enc.reshape(1, _H)
    bd = b_dec.reshape(1, _D)
    return _chunk(x, W_enc, be, W_dec, bd)


def _chunk(x, W_enc, be, W_dec, bd):
    n = x.shape[0]

    z = pl.pallas_call(
        _encode_body,
        grid=(n // _RT_ENC, _H // _HT),
        in_specs=[
            pl.BlockSpec((_RT_ENC, _D), lambda i, j: (i, 0)),
            pl.BlockSpec((_HT, _D), lambda i, j: (j, 0)),
            pl.BlockSpec((1, _HT), lambda i, j: (0, j)),
        ],
        out_specs=pl.BlockSpec((_RT_ENC, _HT), lambda i, j: (i, j)),
        out_shape=jax.ShapeDtypeStruct((n, _H), jnp.float32),
        compiler_params=pltpu.CompilerParams(
            dimension_semantics=("parallel", "arbitrary"),
        ),
    )(x, W_enc, be)

    t = make_sc_threshold(n)(z).reshape(n, 1)

    zs, xh = pl.pallas_call(
        _decode_body,
        grid=(n // _RT_DEC, _H // _HT),
        in_specs=[
            pl.BlockSpec((_RT_DEC, _HT), lambda i, j: (i, j)),
            pl.BlockSpec((_RT_DEC, 1), lambda i, j: (i, 0)),
            pl.BlockSpec((_D, _HT), lambda i, j: (0, j)),
            pl.BlockSpec((1, _D), lambda i, j: (0, 0)),
        ],
        out_specs=[
            pl.BlockSpec((_RT_DEC, _HT), lambda i, j: (i, j)),
            pl.BlockSpec((_RT_DEC, _D), lambda i, j: (i, 0)),
        ],
        out_shape=[
            jax.ShapeDtypeStruct((n, _H), jnp.float32),
            jax.ShapeDtypeStruct((n, _D), jnp.float32),
        ],
        compiler_params=pltpu.CompilerParams(
            dimension_semantics=("parallel", "arbitrary"),
        ),
    )(z, t, W_dec, bd)
    return (zs, xh)
